# baseline (speedup 1.0000x reference)
import functools

import jax
import jax.numpy as jnp
from jax.experimental import pallas as pl
from jax.experimental.pallas import tpu as pltpu

_BF = jnp.bfloat16
_F32 = jnp.float32
_VMEM_LIMIT = 52 * 1024 * 1024


def _ru(v, m):
    return (v + m - 1) // m * m


def _ceil(a, b):
    return -(-a // b)


def _tile_of(npad, cap):
    t = min(cap, npad)
    while npad % t:
        t -= 128
    return t


_TM_BUDGET = 34 * 1024 * 1024


def _pick_tm(m, bytes_at, tm_min=1024, tm_max=8192):
    best = tm_min
    tm = tm_min
    while tm <= tm_max:
        if bytes_at(tm) <= _TM_BUDGET and _ru(m, tm) - m <= m // 8:
            best = tm
        tm *= 2
    return best


def _rowvec(v, n, dtype=_F32):
    v = v.astype(dtype)
    if v.shape[0] != n:
        v = jnp.pad(v, (0, n - v.shape[0]))
    return v.reshape(1, n)


def _gemm_body(*refs, relu, pre, post, res):
    it = iter(refs)
    a_ref = next(it)
    b_ref = next(it)
    ps_ref = next(it) if pre else None
    pt_ref = next(it) if pre else None
    s_ref = next(it)
    t_ref = next(it)
    qs_ref = next(it) if post else None
    qt_ref = next(it) if post else None
    r_ref = next(it) if res else None
    o_ref = next(it)

    a = a_ref[...]
    if pre:
        a = jnp.maximum(a * ps_ref[...] + pt_ref[...], jnp.zeros((), a.dtype))
    y = jnp.dot(a, b_ref[...], preferred_element_type=_F32)
    y = y * s_ref[...] + t_ref[...]
    if res:
        y = y + r_ref[...].astype(_F32)
    if post:
        y = y * qs_ref[...] + qt_ref[...]
    if relu:
        y = jnp.maximum(y, 0.0)
    o_ref[...] = y.astype(o_ref.dtype)


def _gemm(a, wp, n_out, scale, shift, relu=False, residual=None,
          pre_s=None, pre_t=None, post_s=None, post_t=None):
    m, kdim = a.shape
    kp, npad = wp.shape
    assert kp == _ru(kdim, 128)

    tn = _tile_of(npad, 512)
    if m >= 1024:
        has_res = residual is not None
        tm = _pick_tm(m, lambda t: 2 * (2 * t * kp + 2 * kp * tn
                                        + (4 if has_res else 2) * t * tn))
    else:
        tm = _ru(m, 8)
    mp = _ru(m, tm)
    grid = (npad // tn, mp // tm)

    a_p = a.astype(_BF)
    if a_p.shape != (mp, kp):
        a_p = jnp.pad(a_p, ((0, mp - m), (0, kp - kdim)))

    specs = [
        pl.BlockSpec((tm, kp), lambda j, i: (i, 0)),
        pl.BlockSpec((kp, tn), lambda j, i: (0, j)),
    ]
    args = [a_p, wp]
    if pre_s is not None:
        specs += [pl.BlockSpec((1, kp), lambda j, i: (0, 0))] * 2
        args += [_rowvec(pre_s, kp, _BF), _rowvec(pre_t, kp, _BF)]
    specs += [pl.BlockSpec((1, tn), lambda j, i: (0, j))] * 2
    args += [_rowvec(scale, npad), _rowvec(shift, npad)]
    if post_s is not None:
        specs += [pl.BlockSpec((1, tn), lambda j, i: (0, j))] * 2
        args += [_rowvec(post_s, npad), _rowvec(post_t, npad)]
    if residual is not None:
        specs.append(pl.BlockSpec((tm, tn), lambda j, i: (i, j)))
        r_p = residual.astype(_BF)
        if r_p.shape != (mp, npad):
            r_p = jnp.pad(r_p, ((0, mp - m), (0, npad - r_p.shape[1])))
        args.append(r_p)

    body = functools.partial(_gemm_body, relu=relu, pre=pre_s is not None,
                             post=post_s is not None, res=residual is not None)
    out = pl.pallas_call(
        body,
        out_shape=jax.ShapeDtypeStruct((mp, npad), _BF),
        grid=grid,
        in_specs=specs,
        out_specs=pl.BlockSpec((tm, tn), lambda j, i: (i, j)),
        compiler_params=pltpu.CompilerParams(
            dimension_semantics=("parallel", "parallel"),
            vmem_limit_bytes=_VMEM_LIMIT),
    )(*args)
    if (mp, npad) != (m, n_out):
        out = out[:m, :n_out]
    return out


def _dconv_body(*refs, na, k, wshift, tm, ulen, relu, nkc):
    it = iter(refs)
    a_refs = [next(it) for _ in range(na)]
    w_ref = next(it)
    s_ref = next(it)
    t_ref = next(it)
    o_ref = next(it)
    acc_ref = next(it) if nkc > 1 else None

    blocks = [r[...] for r in a_refs]
    cols = []
    for j in range(k):
        pieces = []
        for bi in range(na):
            lo = max(0, j - bi * tm)
            hi = min(tm, j + ulen - bi * tm)
            if hi > lo:
                pieces.append(blocks[bi][lo:hi])
        cols.append(pieces[0] if len(pieces) == 1
                    else jnp.concatenate(pieces, axis=0))
    u = jnp.concatenate(cols, axis=1)
    part = None
    for kh in range(k):
        r0 = kh * wshift
        d = jnp.dot(u[r0:r0 + tm], w_ref[0, kh], preferred_element_type=_F32)
        part = d if part is None else part + d

    def _epilogue(acc):
        y = acc * s_ref[...] + t_ref[...]
        if relu:
            y = jnp.maximum(y, 0.0)
        o_ref[...] = y.astype(o_ref.dtype)

    if nkc == 1:
        _epilogue(part)
    else:
        kc = pl.program_id(2)

        @pl.when(kc == 0)
        def _():
            acc_ref[...] = part

        @pl.when(kc > 0)
        def _():
            acc_ref[...] = acc_ref[...] + part

        @pl.when(kc == nkc - 1)
        def _():
            _epilogue(acc_ref[...])


def _pack_direct_w(wp, nkc):
    taps, cp, npad = wp.shape
    k = int(round(taps ** 0.5))
    ck = cp // nkc
    w = wp.reshape(k, k, nkc, ck, npad)
    if nkc > 1:
        w = jnp.transpose(w, (2, 0, 1, 3, 4))
    else:
        w = w.reshape(1, k, k, ck, npad)
    return w.reshape(nkc, k, k * ck, npad)


def _conv_direct(x, w4, k, pad, scale, shift, relu, out_ch):
    n, h, w_dim, c = x.shape
    nkc, _, kck, npad = w4.shape
    ck = kck // k
    cp = nkc * ck
    if isinstance(pad, int):
        pad = (pad, pad, pad, pad)
    p_t, p_b, p_l, p_r = pad
    wo = w_dim + p_l + p_r - k + 1
    p_r += (-(w_dim + p_l + p_r)) % 8
    hp, wq = h + p_t + p_b, w_dim + p_l + p_r
    ho = hp - k + 1
    m = n * hp * wq
    ms = (k - 1) * wq + (k - 1)

    tn = _tile_of(npad, 512)
    tm_min = 1024
    while ms > tm_min:
        tm_min *= 2

    def _dc_bytes(t):
        return 2 * (2 * 2 * t * ck
                    + 2 * k * kck * tn
                    + 2 * t * tn
                    + (4 * t * tn if nkc > 1 else 0)) + (t + ms) * kck * 2

    tm = _pick_tm(m, _dc_bytes, tm_min=tm_min)
    ulen = tm + (k - 1) * wq
    na = _ceil(tm + ms, tm)
    nb = _ceil(m, tm)
    rows = (nb + na - 1) * tm

    xp = jnp.pad(x.astype(_BF),
                 ((0, 0), (p_t, p_b), (p_l, p_r), (0, cp - c)))
    af = xp.reshape(m, cp)
    af = jnp.pad(af, ((0, rows - m), (0, 0)))

    g3 = nkc > 1
    if g3:
        grid = (npad // tn, nb, nkc)
        a_specs = [pl.BlockSpec((tm, ck), (lambda j, i, kc, d=d: (i + d, kc)))
                   for d in range(na)]
        w_spec = pl.BlockSpec((1, k, kck, tn), lambda j, i, kc: (kc, 0, 0, j))
        v_spec = pl.BlockSpec((1, tn), lambda j, i, kc: (0, j))
        o_spec = pl.BlockSpec((tm, tn), lambda j, i, kc: (i, j))
        sems = ("parallel", "parallel", "arbitrary")
        scratch = [pltpu.VMEM((tm, tn), _F32)]
    else:
        grid = (npad // tn, nb)
        a_specs = [pl.BlockSpec((tm, ck), (lambda j, i, d=d: (i + d, 0)))
                   for d in range(na)]
        w_spec = pl.BlockSpec((1, k, kck, tn), lambda j, i: (0, 0, 0, j))
        v_spec = pl.BlockSpec((1, tn), lambda j, i: (0, j))
        o_spec = pl.BlockSpec((tm, tn), lambda j, i: (i, j))
        sems = ("parallel", "parallel")
        scratch = []

    body = functools.partial(_dconv_body, na=na, k=k, wshift=wq, tm=tm,
                             ulen=ulen, relu=relu, nkc=nkc)
    out = pl.pallas_call(
        body,
        out_shape=jax.ShapeDtypeStruct((nb * tm, npad), _BF),
        grid=grid,
        in_specs=[*a_specs, w_spec, v_spec, v_spec],
        out_specs=o_spec,
        scratch_shapes=scratch,
        compiler_params=pltpu.CompilerParams(
            dimension_semantics=sems,
            vmem_limit_bytes=_VMEM_LIMIT),
    )(af, *([af] * (na - 1)), w4, _rowvec(scale, npad), _rowvec(shift, npad))
    return out[:m].reshape(n, hp, wq, npad)[:, :ho, :wo, :out_ch]


def _phase_split(x):
    n, h, w, c = x.shape
    return x.reshape(n, h // 2, 2, w // 2, 2, c).transpose(
        0, 1, 3, 2, 4, 5).reshape(n, h // 2, w // 2, 4 * c)


def _s2_w_phase(wp, cin):
    npad = wp.shape[1]
    w9 = wp[:9 * cin].reshape(3, 3, cin, npad)
    zero = jnp.zeros((cin, npad), wp.dtype)
    taps = []
    for kh in (0, 1):
        for kw in (0, 1):
            rows = []
            for a in (0, 1):
                for b in (0, 1):
                    i, j = 2 * kh + a - 1, 2 * kw + b - 1
                    rows.append(w9[i, j] if 0 <= i < 3 and 0 <= j < 3 else zero)
            taps.append(jnp.concatenate(rows, axis=0))
    return jnp.stack(taps, axis=0)


def _conv3_s2_phase(xp4, cp, bn, relu):
    cin = (xp4.shape[-1]) // 4
    w2 = _s2_w_phase(cp["wp"], cin)
    nkc = 2 if w2.shape[1] > 512 else 1
    w4 = _pack_direct_w(w2, nkc)
    scale, shift = bn["scale"], bn["shift"]
    return _conv_direct(xp4, w4, 2, (1, 0, 1, 0), scale, shift, relu, cp["O"])


def _w5_phase(wp, cin, cout):
    cp_, npad = wp.shape[1], wp.shape[2]
    w25 = wp.reshape(5, 5, cp_, npad)[:, :, :cin, :cout]
    zero = jnp.zeros((cin, cout), wp.dtype)
    taps = []
    for dy in (-1, 0, 1):
        for dx in (-1, 0, 1):
            col_groups = []
            for a in (0, 1):
                for b in (0, 1):
                    rows = []
                    for ap in (0, 1):
                        for bp in (0, 1):
                            i = 2 * dy + ap - a + 2
                            j = 2 * dx + bp - b + 2
                            rows.append(w25[i, j] if 0 <= i < 5 and 0 <= j < 5
                                        else zero)
                    col_groups.append(jnp.concatenate(rows, axis=0))
            taps.append(jnp.concatenate(col_groups, axis=1))
    return jnp.stack(taps, axis=0)


def _conv5_phase(x_ph, cp, cin, n_low, h_low, w_low):
    cout = cp["O"]
    w3 = _w5_phase(cp["wp"], cin, cout)
    nkc = 2 if w3.shape[1] > 512 else 1
    w4 = _pack_direct_w(w3, nkc)
    x4 = x_ph.reshape(n_low, h_low, w_low, 4 * cin)
    return _conv_direct(x4, w4, 3, 1, jnp.ones((4 * cout,), _F32),
                        jnp.zeros((4 * cout,), _F32), False, 4 * cout)


def _conv1_s2_phase(xp4, cp):
    n, hh, wh, c4 = xp4.shape
    cin = c4 // 4
    wsc = jnp.pad(cp["wp"][:cin], ((0, c4 - cin), (0, 0)))
    o_ch = cp["O"]
    out = _gemm(xp4.reshape(n * hh * wh, c4), wsc, o_ch,
                jnp.ones((o_ch,), _F32), jnp.zeros((o_ch,), _F32))
    return out.reshape(n, hh, wh, o_ch)


def _patches(x, k, stride, pad):
    n, h, w, c = x.shape
    ho = (h + 2 * pad - k) // stride + 1
    wo = (w + 2 * pad - k) // stride + 1
    xp = jnp.pad(x, ((0, 0), (pad, pad), (pad, pad), (0, 0)))
    cols = [xp[:, i:i + stride * (ho - 1) + 1:stride,
               j:j + stride * (wo - 1) + 1:stride, :]
            for i in range(k) for j in range(k)]
    return jnp.concatenate(cols, axis=-1).reshape(n * ho * wo, k * k * c), (n, ho, wo)


def _conv(x, cp, stride=1, pad=0, bn=None, relu=False, residual=None,
          pre_bn=None, post_bn=None):
    o_ch, k, mode = cp["O"], cp["k"], cp["mode"]
    if bn is not None:
        scale, shift = bn["scale"], bn["shift"]
    else:
        scale = jnp.ones((o_ch,), _F32)
        shift = jnp.zeros((o_ch,), _F32)
    if cp["b"] is not None:
        shift = shift + scale * cp["b"]

    if mode == "direct":
        nkc = 2 if cp["wp"].shape[1] > 512 else 1
        w4 = _pack_direct_w(cp["wp"], nkc)
        return _conv_direct(x, w4, k, pad, scale, shift, relu, o_ch)

    ps = pt = qs = qt = None
    if pre_bn is not None:
        ps, pt = pre_bn["scale"], pre_bn["shift"]
    if post_bn is not None:
        qs, qt = post_bn["scale"], post_bn["shift"]

    if k == 1:
        if stride != 1:
            x = x[:, ::stride, ::stride, :]
        n, ho, wo, c = x.shape
        a = x.reshape(n * ho * wo, c)
    else:
        a, (n, ho, wo) = _patches(x, k, stride, pad)

    res = residual.reshape(n * ho * wo, o_ch) if residual is not None else None
    out = _gemm(a, cp["wp"], o_ch, scale, shift, relu=relu, residual=res,
                pre_s=ps, pre_t=pt, post_s=qs, post_t=qt)
    return out.reshape(n, ho, wo, o_ch)


def _up2(x):
    n, h, w, c = x.shape
    return jnp.broadcast_to(x[:, :, None, :, None, :],
                            (n, h, 2, w, 2, c)).reshape(n, 2 * h, 2 * w, c)


class _Slot:
    pass


_S = _Slot()


def _conv_p(cin, cout, k, bias=True, stride=1):
    if k == 1:
        mode = "mat"
    elif stride == 1 and cin >= 64:
        mode = "direct"
    else:
        mode = "im2col"
    return {"b": _S if bias else None, "k": k, "stride": stride,
            "O": cout, "mode": mode, "wp": _S}


def _bn_p():
    return {"scale": _S, "shift": _S}


def _res_unit_p(cin, cout, stride):
    mid = cout // 4
    sc = _conv_p(cin, cout, 1, bias=False) if (stride != 1 or cin != cout) else None
    return {"shortcut": sc,
            "conv1": _conv_p(cin, mid, 1, bias=False), "bn1": _bn_p(),
            "conv2": _conv_p(mid, mid, 3, bias=False, stride=stride),
            "bn2": _bn_p(),
            "conv3": _conv_p(mid, cout, 1, bias=False)}


def _res_block_p(cin, cout, stride, n_units):
    units = [_res_unit_p(cin, cout, stride)]
    post = [None]
    for _ in range(n_units - 1):
        units.append(_res_unit_p(cout, cout, 1))
        post.append(_bn_p())
    return {"units": units, "post_bn": post, "stride": stride}


def _dense_unit_p(cin):
    return {"bn1": _bn_p(), "conv1": _conv_p(cin, 128, 1),
            "bn2": _bn_p(), "conv2": _conv_p(128, 32, 5)}


def _dense_block_p(cin, n_units):
    return {"units": [_dense_unit_p(cin + 32 * i) for i in range(n_units)],
            "final_bn": _bn_p()}


def _decoder_p():
    return {"conv1": _conv_p(1024, 256, 5, bias=False),
            "dense1": _dense_block_p(256, 8),
            "conv2": _conv_p(512, 512, 1, bias=False),
            "conv3": _conv_p(512, 128, 5, bias=False),
            "dense2": _dense_block_p(128, 4),
            "conv4": _conv_p(256, 256, 1, bias=False),
            "conv5": _conv_p(256, 64, 5, bias=False)}


def _encoder_p():
    return {"conv1": _conv_p(3, 64, 7), "bn1": _bn_p(),
            "block1": _res_block_p(64, 256, 1, 3),
            "block2": _res_block_p(256, 512, 2, 4),
            "block3": _res_block_p(512, 1024, 2, 6),
            "block4": _res_block_p(1024, 2048, 2, 3),
            "conv2": _conv_p(2048, 1024, 1)}


def _net_p():
    return {"encoder": _encoder_p(),
            "np_branch": _decoder_p(), "np_head": _conv_p(64, 2, 1),
            "hv_branch": _decoder_p(), "hv_head": _conv_p(64, 2, 1),
            "nc_branch": _decoder_p(), "nc_head": _conv_p(64, 3, 1),
            "dec1_branches": ["np_branch", "hv_branch", "nc_branch"],
            "dec1_wp": _S}


def _res_unit(x, p, stride, post_bn):
    if stride == 2:
        skip = _conv1_s2_phase(_phase_split(x), p["shortcut"])
        out = _conv(x, p["conv1"], bn=p["bn1"], relu=True)
        out = _conv3_s2_phase(_phase_split(out), p["conv2"], p["bn2"], True)
    else:
        skip = _conv(x, p["shortcut"]) if p["shortcut"] is not None else x
        out = _conv(x, p["conv1"], bn=p["bn1"], relu=True)
        out = _conv(out, p["conv2"], pad=1, bn=p["bn2"], relu=True)
    return _conv(out, p["conv3"], residual=skip, post_bn=post_bn,
                 relu=post_bn is not None)


def _res_block(x, p):
    for idx, (u, pbn) in enumerate(zip(p["units"], p["post_bn"])):
        x = _res_unit(x, u, p["stride"] if idx == 0 else 1, pbn)
    return x


def _dense_block(x, p):
    for u in p["units"]:
        mid = _conv(x, u["conv1"], pre_bn=u["bn1"], bn=u["bn2"], relu=True)
        mid = _conv(mid, u["conv2"], pad=2)
        x = jnp.concatenate([x, mid], axis=-1)
    return x


def _up_skip_gemm(x, cp, pre_bn, skip_ph):
    n, hh, ww, cd = x.shape
    o = cp["O"]
    out = _gemm(x.reshape(n * hh * ww, cd), jnp.tile(cp["wp"], (1, 4)), 4 * o,
                jnp.ones((4 * o,), _F32), jnp.zeros((4 * o,), _F32),
                residual=skip_ph, pre_s=pre_bn["scale"], pre_t=pre_bn["shift"])
    return out, (n, hh, ww)


def _decoder(b1_ph, b2_ph, p, conv1_out):
    out = _dense_block(conv1_out, p["dense1"])
    ph, (n, hh, ww) = _up_skip_gemm(out, p["conv2"], p["dense1"]["final_bn"],
                                    b2_ph)
    out = _conv5_phase(ph, p["conv3"], p["conv2"]["O"], n, hh, ww)
    o3 = p["conv3"]["O"]
    out = out.reshape(n, hh, ww, 2, 2, o3).transpose(
        0, 1, 3, 2, 4, 5).reshape(n, 2 * hh, 2 * ww, o3)
    out = _dense_block(out, p["dense2"])
    ph, (n, hh, ww) = _up_skip_gemm(out, p["conv4"], p["dense2"]["final_bn"],
                                    b1_ph)
    return _conv5_phase(ph, p["conv5"], p["conv4"]["O"], n, hh, ww), (n, hh, ww)


def kernel(*args):
    ws, x = list(args[:-1]), args[-1]
    skeleton = _net_p()
    leaves, treedef = jax.tree_util.tree_flatten(skeleton)
    w_it = iter(ws)
    filled = [next(w_it) if isinstance(l, _Slot) else l for l in leaves]
    params = jax.tree_util.tree_unflatten(treedef, filled)

    xh = jnp.transpose(x.astype(_F32), (0, 2, 3, 1)).astype(_BF)

    ep = params["encoder"]
    out1 = _conv(xh, ep["conv1"], pad=3, bn=ep["bn1"], relu=True)
    out1 = _res_block(out1, ep["block1"])
    out2 = _res_block(out1, ep["block2"])
    out3 = _res_block(out2, ep["block3"])
    out4 = _res_block(out3, ep["block4"])
    out4 = _conv(out4, ep["conv2"])
    enc = [out1, out2, out3, out4]

    branches = params["dec1_branches"]
    nb = len(branches)
    dec_in = _up2(out4) + out3
    w4 = _pack_direct_w(params["dec1_wp"], 4)
    fused = _conv_direct(dec_in, w4, 5, 2,
                         jnp.ones((256 * nb,), _F32),
                         jnp.zeros((256 * nb,), _F32), False, 256 * nb)

    b1_ph = _phase_split(out1)
    b1_ph = b1_ph.reshape(-1, b1_ph.shape[-1])
    b2_ph = _phase_split(out2)
    b2_ph = b2_ph.reshape(-1, b2_ph.shape[-1])

    heads = {"np_branch": "np_head", "hv_branch": "hv_head",
             "nc_branch": "nc_head"}
    outputs = []
    for bi, bname in enumerate(branches):
        dec_ph, (n, hh, ww) = _decoder(b1_ph, b2_ph, params[bname],
                                       fused[..., bi * 256:(bi + 1) * 256])
        hcp = params[heads[bname]]
        c5 = params[bname]["conv5"]["O"]
        dec = dec_ph.reshape(n, hh, ww, 2, 2, c5).transpose(
            0, 1, 3, 2, 4, 5).reshape(n, 2 * hh, 2 * ww, c5)
        ho = hcp["O"]
        hout = _gemm(dec.reshape(n * 2 * hh * 2 * ww, c5), hcp["wp"], ho,
                     jnp.ones((ho,), _F32), hcp["b"].astype(_F32))
        hout = hout.reshape(n, 2 * hh, 2 * ww, ho)
        outputs.append(jnp.transpose(hout, (0, 3, 1, 2)).astype(_F32))
    return outputs


# speedup vs baseline: 1.0027x; 1.0027x over previous
import functools

import jax
import jax.numpy as jnp
from jax.experimental import pallas as pl
from jax.experimental.pallas import tpu as pltpu

_BF = jnp.bfloat16
_F32 = jnp.float32
_VMEM_LIMIT = 52 * 1024 * 1024


def _ru(v, m):
    return (v + m - 1) // m * m


def _ceil(a, b):
    return -(-a // b)


def _tile_of(npad, cap):
    t = min(cap, npad)
    while npad % t:
        t -= 128
    return t


_TM_BUDGET = 34 * 1024 * 1024


def _pick_tm(m, bytes_at, tm_min=1024, tm_max=8192):
    best = tm_min
    tm = tm_min
    while tm <= tm_max:
        if bytes_at(tm) <= _TM_BUDGET and _ru(m, tm) - m <= m // 8:
            best = tm
        tm *= 2
    return best


def _rowvec(v, n, dtype=_F32):
    v = v.astype(dtype)
    if v.shape[0] != n:
        v = jnp.pad(v, (0, n - v.shape[0]))
    return v.reshape(1, n)


def _gemm_body(*refs, relu, pre, post, res):
    it = iter(refs)
    a_ref = next(it)
    b_ref = next(it)
    ps_ref = next(it) if pre else None
    pt_ref = next(it) if pre else None
    s_ref = next(it)
    t_ref = next(it)
    qs_ref = next(it) if post else None
    qt_ref = next(it) if post else None
    r_ref = next(it) if res else None
    o_ref = next(it)

    a = a_ref[...]
    if pre:
        a = jnp.maximum(a * ps_ref[...] + pt_ref[...], jnp.zeros((), a.dtype))
    y = jnp.dot(a, b_ref[...], preferred_element_type=_F32)
    y = y * s_ref[...] + t_ref[...]
    if res:
        y = y + r_ref[...].astype(_F32)
    if post:
        y = y * qs_ref[...] + qt_ref[...]
    if relu:
        y = jnp.maximum(y, 0.0)
    o_ref[...] = y.astype(o_ref.dtype)


def _gemm(a, wp, n_out, scale, shift, relu=False, residual=None,
          pre_s=None, pre_t=None, post_s=None, post_t=None):
    m, kdim = a.shape
    kp, npad = wp.shape
    assert kp == _ru(kdim, 128)

    tn = _tile_of(npad, 512)
    if m >= 1024:
        has_res = residual is not None
        tm = _pick_tm(m, lambda t: 2 * (2 * t * kp + 2 * kp * tn
                                        + (4 if has_res else 2) * t * tn))
    else:
        tm = _ru(m, 8)
    mp = _ru(m, tm)
    grid = (npad // tn, mp // tm)

    a_p = a.astype(_BF)
    if a_p.shape != (mp, kp):
        a_p = jnp.pad(a_p, ((0, mp - m), (0, kp - kdim)))

    specs = [
        pl.BlockSpec((tm, kp), lambda j, i: (i, 0)),
        pl.BlockSpec((kp, tn), lambda j, i: (0, j)),
    ]
    args = [a_p, wp]
    if pre_s is not None:
        specs += [pl.BlockSpec((1, kp), lambda j, i: (0, 0))] * 2
        args += [_rowvec(pre_s, kp, _BF), _rowvec(pre_t, kp, _BF)]
    specs += [pl.BlockSpec((1, tn), lambda j, i: (0, j))] * 2
    args += [_rowvec(scale, npad), _rowvec(shift, npad)]
    if post_s is not None:
        specs += [pl.BlockSpec((1, tn), lambda j, i: (0, j))] * 2
        args += [_rowvec(post_s, npad), _rowvec(post_t, npad)]
    if residual is not None:
        specs.append(pl.BlockSpec((tm, tn), lambda j, i: (i, j)))
        r_p = residual.astype(_BF)
        if r_p.shape != (mp, npad):
            r_p = jnp.pad(r_p, ((0, mp - m), (0, npad - r_p.shape[1])))
        args.append(r_p)

    body = functools.partial(_gemm_body, relu=relu, pre=pre_s is not None,
                             post=post_s is not None, res=residual is not None)
    out = pl.pallas_call(
        body,
        out_shape=jax.ShapeDtypeStruct((mp, npad), _BF),
        grid=grid,
        in_specs=specs,
        out_specs=pl.BlockSpec((tm, tn), lambda j, i: (i, j)),
        compiler_params=pltpu.CompilerParams(
            dimension_semantics=("parallel", "parallel"),
            vmem_limit_bytes=_VMEM_LIMIT),
    )(*args)
    if (mp, npad) != (m, n_out):
        out = out[:m, :n_out]
    return out


def _dconv_body(*refs, na, k, wshift, tm, ulen, relu, nkc):
    it = iter(refs)
    a_refs = [next(it) for _ in range(na)]
    w_ref = next(it)
    s_ref = next(it)
    t_ref = next(it)
    o_ref = next(it)
    acc_ref = next(it) if nkc > 1 else None

    blocks = [r[...] for r in a_refs]
    cols = []
    for j in range(k):
        pieces = []
        for bi in range(na):
            lo = max(0, j - bi * tm)
            hi = min(tm, j + ulen - bi * tm)
            if hi > lo:
                pieces.append(blocks[bi][lo:hi])
        cols.append(pieces[0] if len(pieces) == 1
                    else jnp.concatenate(pieces, axis=0))
    u = jnp.concatenate(cols, axis=1)
    part = None
    for kh in range(k):
        r0 = kh * wshift
        d = jnp.dot(u[r0:r0 + tm], w_ref[0, kh], preferred_element_type=_F32)
        part = d if part is None else part + d

    def _epilogue(acc):
        y = acc * s_ref[...] + t_ref[...]
        if relu:
            y = jnp.maximum(y, 0.0)
        o_ref[...] = y.astype(o_ref.dtype)

    if nkc == 1:
        _epilogue(part)
    else:
        kc = pl.program_id(2)

        @pl.when(kc == 0)
        def _():
            acc_ref[...] = part

        @pl.when(kc > 0)
        def _():
            acc_ref[...] = acc_ref[...] + part

        @pl.when(kc == nkc - 1)
        def _():
            _epilogue(acc_ref[...])


def _pack_direct_w(wp, nkc):
    taps, cp, npad = wp.shape
    k = int(round(taps ** 0.5))
    ck = cp // nkc
    w = wp.reshape(k, k, nkc, ck, npad)
    if nkc > 1:
        w = jnp.transpose(w, (2, 0, 1, 3, 4))
    else:
        w = w.reshape(1, k, k, ck, npad)
    return w.reshape(nkc, k, k * ck, npad)


def _conv_direct(x, w4, k, pad, scale, shift, relu, out_ch):
    n, h, w_dim, c = x.shape
    nkc, _, kck, npad = w4.shape
    ck = kck // k
    cp = nkc * ck
    if isinstance(pad, int):
        pad = (pad, pad, pad, pad)
    p_t, p_b, p_l, p_r = pad
    wo = w_dim + p_l + p_r - k + 1
    p_r += (-(w_dim + p_l + p_r)) % 8
    hp, wq = h + p_t + p_b, w_dim + p_l + p_r
    ho = hp - k + 1
    m = n * hp * wq
    ms = (k - 1) * wq + (k - 1)

    tn = _tile_of(npad, 512)
    tm_min = 1024
    while ms > tm_min:
        tm_min *= 2

    def _dc_bytes(t):
        return 2 * (2 * 2 * t * ck
                    + 2 * k * kck * tn
                    + 2 * t * tn
                    + (4 * t * tn if nkc > 1 else 0)) + (t + ms) * kck * 2

    tm = _pick_tm(m, _dc_bytes, tm_min=tm_min)
    ulen = tm + (k - 1) * wq
    na = _ceil(tm + ms, tm)
    nb = _ceil(m, tm)
    rows = (nb + na - 1) * tm

    xp = jnp.pad(x.astype(_BF),
                 ((0, 0), (p_t, p_b), (p_l, p_r), (0, cp - c)))
    af = xp.reshape(m, cp)
    af = jnp.pad(af, ((0, rows - m), (0, 0)))

    g3 = nkc > 1
    if g3:
        grid = (npad // tn, nb, nkc)
        a_specs = [pl.BlockSpec((tm, ck), (lambda j, i, kc, d=d: (i + d, kc)))
                   for d in range(na)]
        w_spec = pl.BlockSpec((1, k, kck, tn), lambda j, i, kc: (kc, 0, 0, j))
        v_spec = pl.BlockSpec((1, tn), lambda j, i, kc: (0, j))
        o_spec = pl.BlockSpec((tm, tn), lambda j, i, kc: (i, j))
        sems = ("parallel", "parallel", "arbitrary")
        scratch = [pltpu.VMEM((tm, tn), _F32)]
    else:
        grid = (npad // tn, nb)
        a_specs = [pl.BlockSpec((tm, ck), (lambda j, i, d=d: (i + d, 0)))
                   for d in range(na)]
        w_spec = pl.BlockSpec((1, k, kck, tn), lambda j, i: (0, 0, 0, j))
        v_spec = pl.BlockSpec((1, tn), lambda j, i: (0, j))
        o_spec = pl.BlockSpec((tm, tn), lambda j, i: (i, j))
        sems = ("parallel", "parallel")
        scratch = []

    body = functools.partial(_dconv_body, na=na, k=k, wshift=wq, tm=tm,
                             ulen=ulen, relu=relu, nkc=nkc)
    out = pl.pallas_call(
        body,
        out_shape=jax.ShapeDtypeStruct((nb * tm, npad), _BF),
        grid=grid,
        in_specs=[*a_specs, w_spec, v_spec, v_spec],
        out_specs=o_spec,
        scratch_shapes=scratch,
        compiler_params=pltpu.CompilerParams(
            dimension_semantics=sems,
            vmem_limit_bytes=_VMEM_LIMIT),
    )(af, *([af] * (na - 1)), w4, _rowvec(scale, npad), _rowvec(shift, npad))
    return out[:m].reshape(n, hp, wq, npad)[:, :ho, :wo, :out_ch]


def _phase_split(x):
    n, h, w, c = x.shape
    return x.reshape(n, h // 2, 2, w // 2, 2, c).transpose(
        0, 1, 3, 2, 4, 5).reshape(n, h // 2, w // 2, 4 * c)


def _s2_w_phase(wp, cin):
    npad = wp.shape[1]
    w9 = wp[:9 * cin].reshape(3, 3, cin, npad)
    zero = jnp.zeros((cin, npad), wp.dtype)
    taps = []
    for kh in (0, 1):
        for kw in (0, 1):
            rows = []
            for a in (0, 1):
                for b in (0, 1):
                    i, j = 2 * kh + a - 1, 2 * kw + b - 1
                    rows.append(w9[i, j] if 0 <= i < 3 and 0 <= j < 3 else zero)
            taps.append(jnp.concatenate(rows, axis=0))
    return jnp.stack(taps, axis=0)


def _conv3_s2_phase(xp4, cp, bn, relu):
    cin = (xp4.shape[-1]) // 4
    w2 = _s2_w_phase(cp["wp"], cin)
    nkc = 2 if w2.shape[1] > 512 else 1
    w4 = _pack_direct_w(w2, nkc)
    scale, shift = bn["scale"], bn["shift"]
    return _conv_direct(xp4, w4, 2, (1, 0, 1, 0), scale, shift, relu, cp["O"])


def _w5_phase(wp, cin, cout):
    cp_, npad = wp.shape[1], wp.shape[2]
    w25 = wp.reshape(5, 5, cp_, npad)[:, :, :cin, :cout]
    zero = jnp.zeros((cin, cout), wp.dtype)
    taps = []
    for dy in (-1, 0, 1):
        for dx in (-1, 0, 1):
            col_groups = []
            for a in (0, 1):
                for b in (0, 1):
                    rows = []
                    for ap in (0, 1):
                        for bp in (0, 1):
                            i = 2 * dy + ap - a + 2
                            j = 2 * dx + bp - b + 2
                            rows.append(w25[i, j] if 0 <= i < 5 and 0 <= j < 5
                                        else zero)
                    col_groups.append(jnp.concatenate(rows, axis=0))
            taps.append(jnp.concatenate(col_groups, axis=1))
    return jnp.stack(taps, axis=0)


def _conv5_phase(x_ph, cp, cin, n_low, h_low, w_low):
    cout = cp["O"]
    w3 = _w5_phase(cp["wp"], cin, cout)
    nkc = 2 if w3.shape[1] > 512 else 1
    w4 = _pack_direct_w(w3, nkc)
    x4 = x_ph.reshape(n_low, h_low, w_low, 4 * cin)
    return _conv_direct(x4, w4, 3, 1, jnp.ones((4 * cout,), _F32),
                        jnp.zeros((4 * cout,), _F32), False, 4 * cout)


def _conv1_s2_phase(xp4, cp):
    n, hh, wh, c4 = xp4.shape
    cin = c4 // 4
    wsc = jnp.pad(cp["wp"][:cin], ((0, c4 - cin), (0, 0)))
    o_ch = cp["O"]
    out = _gemm(xp4.reshape(n * hh * wh, c4), wsc, o_ch,
                jnp.ones((o_ch,), _F32), jnp.zeros((o_ch,), _F32))
    return out.reshape(n, hh, wh, o_ch)


def _patches(x, k, stride, pad):
    n, h, w, c = x.shape
    ho = (h + 2 * pad - k) // stride + 1
    wo = (w + 2 * pad - k) // stride + 1
    xp = jnp.pad(x, ((0, 0), (pad, pad), (pad, pad), (0, 0)))
    cols = [xp[:, i:i + stride * (ho - 1) + 1:stride,
               j:j + stride * (wo - 1) + 1:stride, :]
            for i in range(k) for j in range(k)]
    return jnp.concatenate(cols, axis=-1).reshape(n * ho * wo, k * k * c), (n, ho, wo)


def _conv(x, cp, stride=1, pad=0, bn=None, relu=False, residual=None,
          pre_bn=None, post_bn=None):
    o_ch, k, mode = cp["O"], cp["k"], cp["mode"]
    if bn is not None:
        scale, shift = bn["scale"], bn["shift"]
    else:
        scale = jnp.ones((o_ch,), _F32)
        shift = jnp.zeros((o_ch,), _F32)
    if cp["b"] is not None:
        shift = shift + scale * cp["b"]

    if mode == "direct":
        nkc = 2 if cp["wp"].shape[1] > 512 else 1
        w4 = _pack_direct_w(cp["wp"], nkc)
        return _conv_direct(x, w4, k, pad, scale, shift, relu, o_ch)

    ps = pt = qs = qt = None
    if pre_bn is not None:
        ps, pt = pre_bn["scale"], pre_bn["shift"]
    if post_bn is not None:
        qs, qt = post_bn["scale"], post_bn["shift"]

    if k == 1:
        if stride != 1:
            x = x[:, ::stride, ::stride, :]
        n, ho, wo, c = x.shape
        a = x.reshape(n * ho * wo, c)
    else:
        a, (n, ho, wo) = _patches(x, k, stride, pad)

    res = residual.reshape(n * ho * wo, o_ch) if residual is not None else None
    out = _gemm(a, cp["wp"], o_ch, scale, shift, relu=relu, residual=res,
                pre_s=ps, pre_t=pt, post_s=qs, post_t=qt)
    return out.reshape(n, ho, wo, o_ch)


def _up2(x):
    n, h, w, c = x.shape
    return jnp.broadcast_to(x[:, :, None, :, None, :],
                            (n, h, 2, w, 2, c)).reshape(n, 2 * h, 2 * w, c)


class _Slot:
    pass


_S = _Slot()


def _conv_p(cin, cout, k, bias=True, stride=1):
    if k == 1:
        mode = "mat"
    elif stride == 1 and cin >= 64:
        mode = "direct"
    else:
        mode = "im2col"
    return {"b": _S if bias else None, "k": k, "stride": stride,
            "O": cout, "mode": mode, "wp": _S}


def _bn_p():
    return {"scale": _S, "shift": _S}


def _res_unit_p(cin, cout, stride):
    mid = cout // 4
    sc = _conv_p(cin, cout, 1, bias=False) if (stride != 1 or cin != cout) else None
    return {"shortcut": sc,
            "conv1": _conv_p(cin, mid, 1, bias=False), "bn1": _bn_p(),
            "conv2": _conv_p(mid, mid, 3, bias=False, stride=stride),
            "bn2": _bn_p(),
            "conv3": _conv_p(mid, cout, 1, bias=False)}


def _res_block_p(cin, cout, stride, n_units):
    units = [_res_unit_p(cin, cout, stride)]
    post = [None]
    for _ in range(n_units - 1):
        units.append(_res_unit_p(cout, cout, 1))
        post.append(_bn_p())
    return {"units": units, "post_bn": post, "stride": stride}


def _dense_unit_p(cin):
    return {"bn1": _bn_p(), "conv1": _conv_p(cin, 128, 1),
            "bn2": _bn_p(), "conv2": _conv_p(128, 32, 5)}


def _dense_block_p(cin, n_units):
    return {"units": [_dense_unit_p(cin + 32 * i) for i in range(n_units)],
            "final_bn": _bn_p()}


def _decoder_p():
    return {"conv1": _conv_p(1024, 256, 5, bias=False),
            "dense1": _dense_block_p(256, 8),
            "conv2": _conv_p(512, 512, 1, bias=False),
            "conv3": _conv_p(512, 128, 5, bias=False),
            "dense2": _dense_block_p(128, 4),
            "conv4": _conv_p(256, 256, 1, bias=False),
            "conv5": _conv_p(256, 64, 5, bias=False)}


def _encoder_p():
    return {"conv1": _conv_p(3, 64, 7), "bn1": _bn_p(),
            "block1": _res_block_p(64, 256, 1, 3),
            "block2": _res_block_p(256, 512, 2, 4),
            "block3": _res_block_p(512, 1024, 2, 6),
            "block4": _res_block_p(1024, 2048, 2, 3),
            "conv2": _conv_p(2048, 1024, 1)}


def _net_p():
    return {"encoder": _encoder_p(),
            "np_branch": _decoder_p(), "np_head": _conv_p(64, 2, 1),
            "hv_branch": _decoder_p(), "hv_head": _conv_p(64, 2, 1),
            "nc_branch": _decoder_p(), "nc_head": _conv_p(64, 3, 1),
            "dec1_branches": ["np_branch", "hv_branch", "nc_branch"],
            "dec1_wp": _S}


def _res_unit(x, p, stride, post_bn):
    if stride == 2:
        skip = _conv1_s2_phase(_phase_split(x), p["shortcut"])
        out = _conv(x, p["conv1"], bn=p["bn1"], relu=True)
        out = _conv3_s2_phase(_phase_split(out), p["conv2"], p["bn2"], True)
    else:
        skip = _conv(x, p["shortcut"]) if p["shortcut"] is not None else x
        out = _conv(x, p["conv1"], bn=p["bn1"], relu=True)
        out = _conv(out, p["conv2"], pad=1, bn=p["bn2"], relu=True)
    return _conv(out, p["conv3"], residual=skip, post_bn=post_bn,
                 relu=post_bn is not None)


def _res_block(x, p):
    for idx, (u, pbn) in enumerate(zip(p["units"], p["post_bn"])):
        x = _res_unit(x, u, p["stride"] if idx == 0 else 1, pbn)
    return x


def _dense_block(x, p):
    for u in p["units"]:
        mid = _conv(x, u["conv1"], pre_bn=u["bn1"], bn=u["bn2"], relu=True)
        mid = _conv(mid, u["conv2"], pad=2)
        x = jnp.concatenate([x, mid], axis=-1)
    return x


def _up_skip_gemm(x, cp, pre_bn, skip_ph):
    n, hh, ww, cd = x.shape
    o = cp["O"]
    out = _gemm(x.reshape(n * hh * ww, cd), jnp.tile(cp["wp"], (1, 4)), 4 * o,
                jnp.ones((4 * o,), _F32), jnp.zeros((4 * o,), _F32),
                residual=skip_ph, pre_s=pre_bn["scale"], pre_t=pre_bn["shift"])
    return out, (n, hh, ww)


def _decoder(b1_ph, b2_ph, p, conv1_out):
    out = _dense_block(conv1_out, p["dense1"])
    ph, (n, hh, ww) = _up_skip_gemm(out, p["conv2"], p["dense1"]["final_bn"],
                                    b2_ph)
    out = _conv5_phase(ph, p["conv3"], p["conv2"]["O"], n, hh, ww)
    o3 = p["conv3"]["O"]
    out = out.reshape(n, hh, ww, 2, 2, o3).transpose(
        0, 1, 3, 2, 4, 5).reshape(n, 2 * hh, 2 * ww, o3)
    out = _dense_block(out, p["dense2"])
    ph, (n, hh, ww) = _up_skip_gemm(out, p["conv4"], p["dense2"]["final_bn"],
                                    b1_ph)
    return _conv5_phase(ph, p["conv5"], p["conv4"]["O"], n, hh, ww), (n, hh, ww)


def kernel(*args):
    ws, x = list(args[:-1]), args[-1]
    skeleton = _net_p()
    leaves, treedef = jax.tree_util.tree_flatten(skeleton)
    w_it = iter(ws)
    filled = [next(w_it) if isinstance(l, _Slot) else l for l in leaves]
    params = jax.tree_util.tree_unflatten(treedef, filled)

    xh = jnp.transpose(x.astype(_F32), (0, 2, 3, 1)).astype(_BF)

    ep = params["encoder"]
    out1 = _conv(xh, ep["conv1"], pad=3, bn=ep["bn1"], relu=True)
    out1 = _res_block(out1, ep["block1"])
    out2 = _res_block(out1, ep["block2"])
    out3 = _res_block(out2, ep["block3"])
    out4 = _res_block(out3, ep["block4"])
    out4 = _conv(out4, ep["conv2"])
    enc = [out1, out2, out3, out4]

    branches = params["dec1_branches"]
    nb = len(branches)
    dec_in = _up2(out4) + out3
    w4 = _pack_direct_w(params["dec1_wp"], 2)
    fused = _conv_direct(dec_in, w4, 5, 2,
                         jnp.ones((256 * nb,), _F32),
                         jnp.zeros((256 * nb,), _F32), False, 256 * nb)

    b1_ph = _phase_split(out1)
    b1_ph = b1_ph.reshape(-1, b1_ph.shape[-1])
    b2_ph = _phase_split(out2)
    b2_ph = b2_ph.reshape(-1, b2_ph.shape[-1])

    heads = {"np_branch": "np_head", "hv_branch": "hv_head",
             "nc_branch": "nc_head"}
    outputs = []
    for bi, bname in enumerate(branches):
        dec_ph, (n, hh, ww) = _decoder(b1_ph, b2_ph, params[bname],
                                       fused[..., bi * 256:(bi + 1) * 256])
        hcp = params[heads[bname]]
        c5 = params[bname]["conv5"]["O"]
        dec = dec_ph.reshape(n, hh, ww, 2, 2, c5).transpose(
            0, 1, 3, 2, 4, 5).reshape(n, 2 * hh, 2 * ww, c5)
        ho = hcp["O"]
        hout = _gemm(dec.reshape(n * 2 * hh * 2 * ww, c5), hcp["wp"], ho,
                     jnp.ones((ho,), _F32), hcp["b"].astype(_F32))
        hout = hout.reshape(n, 2 * hh, 2 * ww, ho)
        outputs.append(jnp.transpose(hout, (0, 3, 1, 2)).astype(_F32))
    return outputs
